# Initial kernel scaffold; baseline (speedup 1.0000x reference)
import jax
import jax.numpy as jnp
from jax.experimental import pallas as pl


def kernel(output, target):
    raise NotImplementedError("write your pallas kernel here")


# speedup vs baseline: 1.1277x; 1.1277x over previous
import jax
import jax.numpy as jnp
from jax.experimental import pallas as pl

_BINS = 64
_EDGES = _BINS + 1
_EPAD = 72
_LANES = 128
_HW = 384 * 384
_ROWS = _HW // _LANES
_PLANES = 6
_HALF_SD = 100.0 / (2 * _BINS)


def _plane_kernel(o_ref, t_ref, loss_ref):
    p = pl.program_id(0)
    dvec = _HALF_SD * jax.lax.broadcasted_iota(jnp.int32, (_EPAD, 1), 0).astype(jnp.float32)

    def edge_sums(x_ref):
        def body(r, acc):
            row = x_ref[0, r, :] * 50.0
            return acc + jnp.tanh(jnp.broadcast_to(row[None, :], (_EPAD, _LANES)) - dvec)

        acc = jax.lax.fori_loop(0, _ROWS, body, jnp.zeros((_EPAD, _LANES), jnp.float32))
        return jnp.sum(acc, axis=1, keepdims=True)

    t_o = edge_sums(o_ref)
    t_t = edge_sums(t_ref)
    d_o = t_o[0:_BINS] - t_o[1:_EDGES]
    d_t = t_t[0:_BINS] - t_t[1:_EDGES]
    partial = 0.5 * jnp.sum(jnp.abs(d_o - d_t))

    @pl.when(p == 0)
    def _():
        loss_ref[...] = jnp.zeros((1, 1), jnp.float32)

    loss_ref[...] += jnp.full((1, 1), partial)

    @pl.when(p == _PLANES - 1)
    def _():
        loss_ref[...] = loss_ref[...] * (1.0 / (_PLANES * _BINS * _HW))


@jax.jit
def kernel(output, target):
    o = output.reshape(_PLANES, _ROWS, _LANES)
    t = target.reshape(_PLANES, _ROWS, _LANES)
    loss = pl.pallas_call(
        _plane_kernel,
        grid=(_PLANES,),
        in_specs=[
            pl.BlockSpec((1, _ROWS, _LANES), lambda p: (p, 0, 0)),
            pl.BlockSpec((1, _ROWS, _LANES), lambda p: (p, 0, 0)),
        ],
        out_specs=pl.BlockSpec((1, 1), lambda p: (0, 0)),
        out_shape=jax.ShapeDtypeStruct((1, 1), jnp.float32),
    )(o, t)
    return loss[0, 0]


# speedup vs baseline: 2.5475x; 2.2590x over previous
import jax
import jax.numpy as jnp
from jax.experimental import pallas as pl

_BINS = 64
_EDGES = _BINS + 1
_EPAD = 72
_LANES = 128
_HW = 384 * 384
_ROWS = _HW // _LANES
_PLANES = 6
_HALF_SD = 100.0 / (2 * _BINS)


def _plane_kernel(o_ref, t_ref, loss_ref):
    p = pl.program_id(0)
    dvec = _HALF_SD * jax.lax.broadcasted_iota(jnp.int32, (_EPAD, 1), 0).astype(jnp.float32)

    def edge_sums(x_ref):
        unroll = 8

        def body(k, accs):
            tile = x_ref[0, pl.ds(k * unroll, unroll), :] * 50.0
            new = []
            for u in range(unroll):
                row = tile[u:u + 1, :]
                t = jnp.tanh(jnp.broadcast_to(row, (_EPAD, _LANES)) - dvec)
                new.append(accs[u % 2] + t if u < 2 else new[u - 2] + t)
            return (new[unroll - 2], new[unroll - 1])

        zero = jnp.zeros((_EPAD, _LANES), jnp.float32)
        accs = jax.lax.fori_loop(0, _ROWS // unroll, body, (zero, zero))
        return jnp.sum(accs[0] + accs[1], axis=1, keepdims=True)

    t_o = edge_sums(o_ref)
    t_t = edge_sums(t_ref)
    d_o = t_o[0:_BINS] - t_o[1:_EDGES]
    d_t = t_t[0:_BINS] - t_t[1:_EDGES]
    partial = 0.5 * jnp.sum(jnp.abs(d_o - d_t))

    @pl.when(p == 0)
    def _():
        loss_ref[...] = jnp.zeros((1, 1), jnp.float32)

    loss_ref[...] += jnp.full((1, 1), partial)

    @pl.when(p == _PLANES - 1)
    def _():
        loss_ref[...] = loss_ref[...] * (1.0 / (_PLANES * _BINS * _HW))


@jax.jit
def kernel(output, target):
    o = output.reshape(_PLANES, _ROWS, _LANES)
    t = target.reshape(_PLANES, _ROWS, _LANES)
    loss = pl.pallas_call(
        _plane_kernel,
        grid=(_PLANES,),
        in_specs=[
            pl.BlockSpec((1, _ROWS, _LANES), lambda p: (p, 0, 0)),
            pl.BlockSpec((1, _ROWS, _LANES), lambda p: (p, 0, 0)),
        ],
        out_specs=pl.BlockSpec((1, 1), lambda p: (0, 0)),
        out_shape=jax.ShapeDtypeStruct((1, 1), jnp.float32),
    )(o, t)
    return loss[0, 0]
